# Initial kernel scaffold; baseline (speedup 1.0000x reference)
import jax
import jax.numpy as jnp
from jax.experimental import pallas as pl


def kernel(x, table):
    raise NotImplementedError("write your pallas kernel here")


# speedup vs baseline: 2.9611x; 2.9611x over previous
import functools

import jax
import jax.numpy as jnp
from jax import lax
from jax.experimental import pallas as pl
from jax.experimental.pallas import tpu as pltpu
from jax.experimental.pallas import tpu_sc as plsc

BATCH = 4096
HIST = 50
EMB_DIM = 128

NUM_CORES = 2
NUM_SUBCORES = 16
NW = NUM_CORES * NUM_SUBCORES
ROWS = BATCH * HIST
PER_W = ROWS // NW
CHUNK = 128
NCHUNK = PER_W // CHUNK

_mesh = plsc.VectorSubcoreMesh(core_axis_name="c", subcore_axis_name="s")


@functools.partial(
    pl.kernel,
    out_type=jax.ShapeDtypeStruct((ROWS, EMB_DIM), jnp.float32),
    mesh=_mesh,
    scratch_types=[
        pltpu.VMEM((NCHUNK, CHUNK), jnp.int32),
        pltpu.VMEM((CHUNK, EMB_DIM), jnp.float32),
        pltpu.SemaphoreType.DMA,
    ],
)
def _emb_gather(table_hbm, idx_hbm, out_hbm, idx_v, buf, sem):
    wid = lax.axis_index("s") * NUM_CORES + lax.axis_index("c")
    base = wid * PER_W
    pltpu.sync_copy(idx_hbm.at[wid], idx_v)

    def body(j, carry):
        pltpu.async_copy(table_hbm.at[idx_v.at[j]], buf, sem).wait()
        pltpu.sync_copy(buf, out_hbm.at[pl.ds(base + j * CHUNK, CHUNK)])
        return carry

    lax.fori_loop(0, NCHUNK, body, 0, unroll=False)


def kernel(x, table):
    idx = x.reshape(NW, NCHUNK, CHUNK).astype(jnp.int32)
    out = _emb_gather(table, idx)
    return out.reshape(BATCH, HIST, EMB_DIM)


# speedup vs baseline: 3.3432x; 1.1290x over previous
import functools

import jax
import jax.numpy as jnp
from jax import lax
from jax.experimental import pallas as pl
from jax.experimental.pallas import tpu as pltpu
from jax.experimental.pallas import tpu_sc as plsc

BATCH = 4096
HIST = 50
EMB_DIM = 128

NUM_CORES = 2
NUM_SUBCORES = 16
NW = NUM_CORES * NUM_SUBCORES
ROWS = BATCH * HIST
PER_W = ROWS // NW
CHUNK = 80
NCHUNK = PER_W // CHUNK
GROUP = 4
NGROUP = NCHUNK // GROUP

_mesh = plsc.VectorSubcoreMesh(core_axis_name="c", subcore_axis_name="s")


@functools.partial(
    pl.kernel,
    out_type=jax.ShapeDtypeStruct((ROWS, EMB_DIM), jnp.float32),
    mesh=_mesh,
    scratch_types=[
        pltpu.VMEM((NCHUNK, CHUNK), jnp.int32),
        [pltpu.VMEM((CHUNK, EMB_DIM), jnp.float32) for _ in range(2 * GROUP)],
        [pltpu.SemaphoreType.DMA for _ in range(4)],
    ],
)
def _emb_gather(table_hbm, idx_hbm, out_hbm, idx_v, bufs, sems):
    wid = lax.axis_index("s") * NUM_CORES + lax.axis_index("c")
    base = wid * PER_W
    gsems = sems[:2]
    psems = sems[2:]

    pltpu.sync_copy(idx_hbm.at[wid], idx_v)

    def start_gathers(g, s):
        for b in range(GROUP):
            k = g * GROUP + b
            pltpu.async_copy(
                table_hbm.at[idx_v.at[k]], bufs[s * GROUP + b], gsems[s])

    def wait_gathers(g, s):
        for b in range(GROUP):
            k = g * GROUP + b
            pltpu.make_async_copy(
                table_hbm.at[idx_v.at[k]], bufs[s * GROUP + b], gsems[s]).wait()

    def do_puts(g, s):
        copies = []
        for b in range(GROUP):
            k = g * GROUP + b
            copies.append(pltpu.async_copy(
                bufs[s * GROUP + b],
                out_hbm.at[pl.ds(base + k * CHUNK, CHUNK)], psems[s]))
        for c in copies:
            c.wait()

    start_gathers(0, 0)
    start_gathers(1, 1)

    def body(u, carry):
        g0 = 2 * u
        wait_gathers(g0, 0)
        do_puts(g0, 0)
        start_gathers(g0 + 2, 0)
        wait_gathers(g0 + 1, 1)
        do_puts(g0 + 1, 1)
        start_gathers(g0 + 3, 1)
        return carry

    lax.fori_loop(0, NGROUP // 2 - 1, body, 0, unroll=False)

    wait_gathers(NGROUP - 2, 0)
    do_puts(NGROUP - 2, 0)
    wait_gathers(NGROUP - 1, 1)
    do_puts(NGROUP - 1, 1)


def kernel(x, table):
    idx = x.reshape(NW, NCHUNK, CHUNK).astype(jnp.int32)
    out = _emb_gather(table, idx)
    return out.reshape(BATCH, HIST, EMB_DIM)


# speedup vs baseline: 5.9566x; 1.7817x over previous
import functools

import jax
import jax.numpy as jnp
from jax import lax
from jax.experimental import pallas as pl
from jax.experimental.pallas import tpu as pltpu
from jax.experimental.pallas import tpu_sc as plsc

BATCH = 4096
HIST = 50
EMB_DIM = 128

NUM_CORES = 2
NUM_SUBCORES = 16
NW = NUM_CORES * NUM_SUBCORES
PER_W = BATCH // NW
GROUP = 8
NGROUP = PER_W // GROUP

_mesh = plsc.VectorSubcoreMesh(core_axis_name="c", subcore_axis_name="s")


@functools.partial(
    pl.kernel,
    out_type=jax.ShapeDtypeStruct((BATCH, HIST, EMB_DIM), jnp.float32),
    mesh=_mesh,
    scratch_types=[
        pltpu.VMEM((PER_W, HIST), jnp.int32),
        [pltpu.VMEM((GROUP, HIST, EMB_DIM), jnp.float32) for _ in range(2)],
        [pltpu.SemaphoreType.DMA for _ in range(4)],
    ],
)
def _emb_gather(table_hbm, idx_hbm, out_hbm, idx_v, bufs, sems):
    wid = lax.axis_index("s") * NUM_CORES + lax.axis_index("c")
    base = wid * PER_W
    gsems = sems[:2]
    psems = sems[2:]

    pltpu.sync_copy(idx_hbm.at[pl.ds(base, PER_W)], idx_v)

    def start_gathers(g, s):
        for b in range(GROUP):
            k = g * GROUP + b
            pltpu.async_copy(table_hbm.at[idx_v.at[k]], bufs[s].at[b], gsems[s])

    def wait_gathers(g, s):
        for b in range(GROUP):
            k = g * GROUP + b
            pltpu.make_async_copy(
                table_hbm.at[idx_v.at[k]], bufs[s].at[b], gsems[s]).wait()

    def do_puts(g, s):
        pltpu.async_copy(
            bufs[s], out_hbm.at[pl.ds(base + g * GROUP, GROUP)], psems[s]).wait()

    start_gathers(0, 0)
    start_gathers(1, 1)

    def body(u, carry):
        g0 = 2 * u
        wait_gathers(g0, 0)
        do_puts(g0, 0)
        start_gathers(g0 + 2, 0)
        wait_gathers(g0 + 1, 1)
        do_puts(g0 + 1, 1)
        start_gathers(g0 + 3, 1)
        return carry

    lax.fori_loop(0, NGROUP // 2 - 1, body, 0, unroll=False)

    wait_gathers(NGROUP - 2, 0)
    do_puts(NGROUP - 2, 0)
    wait_gathers(NGROUP - 1, 1)
    do_puts(NGROUP - 1, 1)


def kernel(x, table):
    return _emb_gather(table, x.astype(jnp.int32))
